# baseline (speedup 1.0000x reference)
import functools

import jax
import jax.numpy as jnp
from jax import lax
from jax.experimental import pallas as pl
from jax.experimental.pallas import tpu as pltpu
from jax.experimental.pallas import tpu_sc as plsc

BATCH = 16384
HIST = 50
EMBED = 64
B = BATCH * HIST
NODE = 1000000
NC = 2
NS = 16
NW = NC * NS
BPW = B // NW

RB = 128
NBLK = (NODE + RB - 1) // RB
LINSZ = NBLK * RB
BLK_PER_W = 490

CB = 128
NTC = BATCH // CB // NW
NCH = NTC * HIST
NBUF = 4


def _iota16():
    return lax.iota(jnp.int32, 16)


@jax.jit
def _gather(idx, tt, tail):
    mesh = plsc.VectorSubcoreMesh(core_axis_name="c", subcore_axis_name="s")

    @functools.partial(
        pl.kernel,
        out_type=jax.ShapeDtypeStruct((HIST, EMBED, BATCH), jnp.float32),
        mesh=mesh,
        scratch_types=[
            pltpu.HBM((LINSZ, EMBED), jnp.float32),
            [pltpu.VMEM((EMBED, RB), jnp.float32) for _ in range(NBUF)],
            [pltpu.VMEM((RB, EMBED), jnp.float32) for _ in range(NBUF)],
            pltpu.VMEM((BPW,), jnp.int32),
            [pltpu.VMEM((CB,), jnp.int32) for _ in range(NBUF)],
            pltpu.VMEM(((NODE - (NBLK - 1) * RB) * EMBED,), jnp.float32),
            [pltpu.SemaphoreType.DMA for _ in range(NBUF)],
            [pltpu.SemaphoreType.DMA for _ in range(NBUF)],
            pltpu.SemaphoreType.DMA,
        ],
        compiler_params=pltpu.CompilerParams(needs_layout_passes=False),
    )
    def body(idx_hbm, tt_hbm, tail_hbm, out_hbm, lin_hbm, tv, rv, idx_v, ic,
             tail_v, rsem, wsem, isem):
        tr, rows = tv, rv
        gsem, osem = rsem, wsem
        cidx = lax.axis_index("c")
        sidx = lax.axis_index("s")
        wid = sidx * NC + cidx

        def blk_col(i):
            bi = i * NS + sidx
            valid = bi < NBLK - 1
            r0 = pl.multiple_of(bi * RB, RB)
            return valid, r0

        def start_read(i, b):
            valid, r0 = blk_col(i)

            @pl.when(valid)
            def _():
                for dr in range(0, EMBED, 8):
                    pltpu.async_copy(
                        tt_hbm.at[pl.ds(dr, 8), pl.ds(r0, RB)],
                        tv[b].at[pl.ds(dr, 8), :], rsem[b])

        def transpose_block(b):
            def col(r, carry):
                for c0 in range(0, EMBED, 16):
                    vals = plsc.load_gather(
                        tv[b], [_iota16() + c0, jnp.full((16,), r, jnp.int32)])
                    rv[b][r, pl.ds(c0, 16)] = vals
                return carry

            lax.fori_loop(0, RB, col, 0, unroll=4)

        start_read(0, 0)

        def p1_group(o):
            for j in range(2):
                i = o * 2 + j
                b = j
                bn = (j + 1) % 2

                @pl.when(i + 1 < BLK_PER_W)
                def _():
                    @pl.when(i + 1 >= 2)
                    def _():
                        valid_p, r0_p = blk_col(i - 1)

                        @pl.when(valid_p)
                        def _():
                            pltpu.make_async_copy(
                                rv[bn], lin_hbm.at[pl.ds(r0_p, RB)],
                                wsem[bn]).wait()
                    start_read(i + 1, bn)

                valid, r0 = blk_col(i)

                @pl.when(valid)
                def _():
                    for dr in range(0, EMBED, 8):
                        pltpu.make_async_copy(
                            tt_hbm.at[pl.ds(dr, 8), pl.ds(r0, RB)],
                            tv[b].at[pl.ds(dr, 8), :], rsem[b]).wait()
                    transpose_block(b)
                    pltpu.async_copy(
                        rv[b], lin_hbm.at[pl.ds(r0, RB)], wsem[b])

        pl.loop(0, BLK_PER_W // 2)(p1_group)

        for i in (BLK_PER_W - 2, BLK_PER_W - 1):
            b = i % 2
            valid, r0 = blk_col(i)

            @pl.when(valid)
            def _():
                pltpu.make_async_copy(
                    rv[b], lin_hbm.at[pl.ds(r0, RB)], wsem[b]).wait()

        TAIL0 = (NBLK - 1) * RB
        TAILN = NODE - TAIL0

        @pl.when(sidx == 0)
        def _():
            pltpu.sync_copy(tail_hbm, tail_v)
            for r in range(TAILN):
                for c0 in range(0, EMBED, 16):
                    rv[0][r, pl.ds(c0, 16)] = tail_v[
                        pl.ds(r * EMBED + c0, 16)]
            pltpu.sync_copy(
                rv[0].at[pl.ds(0, TAILN), :],
                lin_hbm.at[pl.ds(TAIL0, TAILN)])

        pltpu.async_copy(idx_hbm.at[pl.ds(wid * BPW, BPW)], idx_v, isem)

        plsc.subcore_barrier()

        pltpu.make_async_copy(
            idx_hbm.at[pl.ds(wid * BPW, BPW)], idx_v, isem).wait()

        def chunk_hc(k):
            tc = k // HIST
            h = k % HIST
            return tc, h

        def build_idx(k, b):
            tc, h = chunk_hc(k)
            off = tc * (CB * HIST) + h
            for q in range(0, CB, 16):
                lin = (_iota16() + q) * HIST + off
                ic[b][pl.ds(q, 16)] = plsc.load_gather(idx_v, [lin])

        def start_gather(k, b):
            build_idx(k, b)
            pltpu.async_copy(lin_hbm.at[ic[b]], rows[b], gsem[b])

        def transpose_chunk(b):
            def col(d, carry):
                for q in range(0, CB, 16):
                    vals = plsc.load_gather(
                        rows[b], [_iota16() + q, jnp.full((16,), d, jnp.int32)])
                    tr[b][d, pl.ds(q, 16)] = vals
                return carry

            lax.fori_loop(0, EMBED, col, 0, unroll=4)

        def start_write(k, b):
            tc, h = chunk_hc(k)
            bcol = pl.multiple_of((wid * NTC + tc) * CB, CB)
            for dr in range(0, EMBED, 8):
                pltpu.async_copy(
                    tr[b].at[pl.ds(dr, 8), :],
                    out_hbm.at[h, pl.ds(dr, 8), pl.ds(bcol, CB)], osem[b])

        def wait_write(k, b):
            tc, h = chunk_hc(k)
            bcol = pl.multiple_of((wid * NTC + tc) * CB, CB)
            for dr in range(0, EMBED, 8):
                pltpu.make_async_copy(
                    tr[b].at[pl.ds(dr, 8), :],
                    out_hbm.at[h, pl.ds(dr, 8), pl.ds(bcol, CB)],
                    osem[b]).wait()

        start_gather(0, 0)

        def p2_group(o):
            for j in range(NBUF):
                k = o * NBUF + j
                b = j
                bn = (j + 1) % NBUF

                @pl.when(k + 1 < NCH)
                def _():
                    @pl.when(k + 1 >= NBUF)
                    def _():
                        wait_write(k + 1 - NBUF, bn)
                    start_gather(k + 1, bn)

                pltpu.make_async_copy(
                    lin_hbm.at[ic[b]], rows[b], gsem[b]).wait()
                transpose_chunk(b)
                start_write(k, b)

        pl.loop(0, NCH // NBUF)(p2_group)

        for k in range(NCH - NBUF, NCH):
            wait_write(k, k % NBUF)

    return body(idx, tt, tail)


def kernel(x, m, table):
    del m
    idx = x.reshape(-1)
    tail = lax.slice(table, ((NBLK - 1) * RB, 0), (NODE, EMBED)).reshape(-1)
    pt = _gather(idx, table.T, tail)
    return pt.transpose(2, 0, 1)


# speedup vs baseline: 3.6452x; 3.6452x over previous
import functools

import jax
import jax.numpy as jnp
from jax import lax
from jax.experimental import pallas as pl
from jax.experimental.pallas import tpu as pltpu
from jax.experimental.pallas import tpu_sc as plsc

BATCH = 16384
HIST = 50
EMBED = 64
B = BATCH * HIST
NODE = 1000000
NC = 2
NS = 16
NW = NC * NS
BPW = B // NW
NBUF = 4
CHUNK = 400
CBATCH = CHUNK // HIST
NCHUNK = BPW // CHUNK


@jax.jit
def _gather(idx, table):
    mesh = plsc.VectorSubcoreMesh(core_axis_name="c", subcore_axis_name="s")

    @functools.partial(
        pl.kernel,
        out_type=jax.ShapeDtypeStruct((BATCH, HIST, EMBED), jnp.float32),
        mesh=mesh,
        scratch_types=[
            pltpu.VMEM((BPW,), jnp.int32),
            [pltpu.VMEM((CHUNK, EMBED), jnp.float32) for _ in range(NBUF)],
            [pltpu.SemaphoreType.DMA for _ in range(NBUF)],
            [pltpu.SemaphoreType.DMA for _ in range(NBUF)],
        ],
        compiler_params=pltpu.CompilerParams(use_tc_tiling_on_sc=False),
    )
    def body(idx_hbm, table_hbm, out_hbm, idx_v, rows, gsem, wsem):
        wid = lax.axis_index("s") * NC + lax.axis_index("c")
        base = wid * BPW
        bbase = wid * (BPW // HIST)

        pltpu.sync_copy(idx_hbm.at[pl.ds(base, BPW)], idx_v)

        def gather_chunk(n, b):
            pltpu.async_copy(
                table_hbm.at[idx_v.at[pl.ds(n * CHUNK, CHUNK)]], rows[b],
                gsem[b])

        def write_chunk(g, b):
            for i in range(CBATCH):
                pltpu.async_copy(
                    rows[b].at[pl.ds(i * HIST, HIST), :],
                    out_hbm.at[bbase + g * CBATCH + i], wsem[b])

        def wait_write(g, b):
            for i in range(CBATCH):
                pltpu.make_async_copy(
                    rows[b].at[pl.ds(i * HIST, HIST), :],
                    out_hbm.at[bbase + g * CBATCH + i], wsem[b]).wait()

        gather_chunk(0, 0)

        def group(o):
            for b in range(NBUF):
                g = o * NBUF + b
                n = g + 1
                bn = (b + 1) % NBUF

                @pl.when(n < NCHUNK)
                def _():
                    @pl.when(n >= NBUF)
                    def _():
                        wait_write(n - NBUF, bn)
                    gather_chunk(n, bn)

                pltpu.make_async_copy(
                    table_hbm.at[idx_v.at[pl.ds(0, CHUNK)]], rows[b],
                    gsem[b]).wait()
                write_chunk(g, b)

        pl.loop(0, NCHUNK // NBUF)(group)

        for g in range(NCHUNK - NBUF, NCHUNK):
            wait_write(g, g % NBUF)

    return body(idx, table)


def kernel(x, m, table):
    del m
    idx = x.reshape(-1)
    tflat = jax.lax.optimization_barrier(table.reshape(-1))
    tlin = tflat.reshape(NODE, EMBED)
    return _gather(idx, tlin)
